# baseline (speedup 1.0000x reference)
import functools

import jax
import jax.numpy as jnp
from jax import lax
from jax.experimental import pallas as pl
from jax.experimental.pallas import tpu as pltpu
from jax.experimental.pallas import tpu_sc as plsc

N = 10000
E = 320000
D = 128

NC = 2
NS = 16
NW = NC * NS
EPW = E // NW
CE = 80
NCH = EPW // CE
NPAD = 10240
ZCH = 16
LANES = 16
NG = CE // LANES
NLOOP = (NCH - 1) // 4

_SPLAT_DNUMS = lax.GatherDimensionNumbers(
    offset_dims=(), collapsed_slice_dims=(0,), start_index_map=(0,))

_MM_BLK = 1000


def _mm_body(x_ref, w_ref, o_ref):
    o_ref[...] = jnp.dot(x_ref[...], w_ref[...],
                         preferred_element_type=jnp.float32)


def _matmul(x, W):
    return pl.pallas_call(
        _mm_body,
        grid=(N // _MM_BLK,),
        in_specs=[
            pl.BlockSpec((_MM_BLK, D), lambda i: (i, 0)),
            pl.BlockSpec((D, D), lambda i: (0, 0)),
        ],
        out_specs=pl.BlockSpec((_MM_BLK, D), lambda i: (i, 0)),
        out_shape=jax.ShapeDtypeStruct((N, D), jnp.float32),
    )(x, W)


def _lane_splat(vec, i):
    idx = jnp.full((LANES, 1), i, dtype=jnp.int32)
    return lax.gather(vec, idx, _SPLAT_DNUMS, (1,),
                      mode=lax.GatherScatterMode.PROMISE_IN_BOUNDS)


def _spmm_body(support_hbm, src_hbm, dst_hbm, w_hbm, out_hbm,
               acc, srcs, dsts, ws, g_rows, s_rows, zbuf,
               sem_i, sem_g, sem_s, sem_z):
    c = lax.axis_index("c")
    s = lax.axis_index("s")
    wid = s * NC + c
    ebase = wid * EPW

    zv = jnp.zeros((LANES,), jnp.float32)
    for r in range(ZCH):
        for j in range(D // LANES):
            zbuf[r, pl.ds(j * LANES, LANES)] = zv

    rows_per_tile = NPAD // NS

    def _zero_fire(i, carry):
        pltpu.async_copy(zbuf,
                         acc.at[pl.ds(s * rows_per_tile + i * ZCH, ZCH)],
                         sem_z)
        return carry

    lax.fori_loop(0, rows_per_tile // ZCH, _zero_fire, 0)

    def pre(ch, k):
        sl = pl.ds(ebase + ch * CE, CE)
        pltpu.async_copy(src_hbm.at[sl], srcs.at[k], sem_i[k])
        pltpu.async_copy(dst_hbm.at[sl], dsts.at[k], sem_i[k])
        pltpu.async_copy(w_hbm.at[sl], ws.at[k], sem_i[k])

    def drain_i(ch, k):
        sl = pl.ds(ebase + ch * CE, CE)
        pltpu.make_async_copy(src_hbm.at[sl], srcs.at[k], sem_i[k]).wait()
        pltpu.make_async_copy(dst_hbm.at[sl], dsts.at[k], sem_i[k]).wait()
        pltpu.make_async_copy(w_hbm.at[sl], ws.at[k], sem_i[k]).wait()

    def fire_gather(x, k):
        pltpu.async_copy(support_hbm.at[srcs.at[k]], g_rows.at[x], sem_g[x])

    def drain_gather(x, k):
        pltpu.make_async_copy(support_hbm.at[srcs.at[k]], g_rows.at[x],
                              sem_g[x]).wait()

    def fire_scatter(x, k):
        pltpu.async_copy(s_rows.at[x], acc.at[dsts.at[k]],
                         sem_s[x], add=True)

    def drain_scatter(x, k):
        pltpu.make_async_copy(s_rows.at[x], acc.at[dsts.at[k]],
                              sem_s[x]).wait()

    def scale(x, k):
        g_x = g_rows.at[x]
        s_x = s_rows.at[x]
        w_k = ws.at[k]

        def _group(g, carry):
            wv = w_k[pl.ds(g * LANES, LANES)]

            def _edge(i, carry2):
                splat = _lane_splat(wv, i)
                r = g * LANES + i
                for j in range(D // LANES):
                    sl = pl.ds(j * LANES, LANES)
                    s_x[r, sl] = g_x[r, sl] * splat
                return carry2

            lax.fori_loop(0, LANES, _edge, 0)
            return carry
        lax.fori_loop(0, NG, _group, 0)

    pre(0, 0)
    pre(1, 1)
    pre(2, 2)
    drain_i(0, 0)
    fire_gather(0, 0)
    drain_i(1, 1)
    fire_gather(1, 1)

    def _zero_drain(i, carry):
        pltpu.make_async_copy(
            zbuf,
            acc.at[pl.ds(s * rows_per_tile + i * ZCH, ZCH)], sem_z).wait()
        return carry

    lax.fori_loop(0, rows_per_tile // ZCH, _zero_drain, 0)
    plsc.subcore_barrier()

    def _quad(p, carry):
        for o in range(4):
            q = 4 * p + o
            x = o % 2
            xp = (o + 1) % 2
            k = o % 4
            kp = (k + 3) % 4
            drain_gather(x, k)
            if o == 0:
                @pl.when(p > 0)
                def _():
                    drain_scatter(xp, kp)
            else:
                drain_scatter(xp, kp)
            scale(x, k)
            if o < 2:
                pre(q + 3, kp)
            else:
                @pl.when(p < NLOOP - 1)
                def _():
                    pre(q + 3, kp)
            if o < 3:
                drain_i(q + 2, (k + 2) % 4)
                fire_gather(x, (k + 2) % 4)
            else:
                @pl.when(p < NLOOP - 1)
                def _():
                    drain_i(q + 2, (k + 2) % 4)
                    fire_gather(x, (k + 2) % 4)
            fire_scatter(x, k)
        return carry

    lax.fori_loop(0, NLOOP, _quad, 0)

    drain_gather(0, 0)
    drain_scatter(1, 3)
    scale(0, 0)
    fire_scatter(0, 0)
    drain_scatter(0, 0)
    plsc.subcore_barrier()

    out_rows = NPAD // NS
    pltpu.sync_copy(acc.at[pl.ds(s * out_rows, out_rows)],
                    out_hbm.at[c, pl.ds(s * out_rows, out_rows)])


def _spmm(support, src, dst, w):
    mesh = plsc.VectorSubcoreMesh(core_axis_name="c", subcore_axis_name="s")
    f = pl.kernel(
        _spmm_body,
        out_type=jax.ShapeDtypeStruct((NC, NPAD, D), jnp.float32),
        mesh=mesh,
        scratch_types=[
            pltpu.VMEM_SHARED((NPAD, D), jnp.float32),
            pltpu.VMEM((4, CE), jnp.int32),
            pltpu.VMEM((4, CE), jnp.int32),
            pltpu.VMEM((4, CE), jnp.float32),
            pltpu.VMEM((2, CE, D), jnp.float32),
            pltpu.VMEM((2, CE, D), jnp.float32),
            pltpu.VMEM((ZCH, D), jnp.float32),
            [pltpu.SemaphoreType.DMA] * 4,
            [pltpu.SemaphoreType.DMA] * 2,
            [pltpu.SemaphoreType.DMA] * 2,
            pltpu.SemaphoreType.DMA,
        ],
    )
    return f(support, src, dst, w)


def _comb_body(p_ref, b_ref, o_ref):
    o_ref[...] = p_ref[0] + p_ref[1] + b_ref[...]


def _combine(partials, b2):
    return pl.pallas_call(
        _comb_body,
        grid=(N // _MM_BLK,),
        in_specs=[
            pl.BlockSpec((NC, _MM_BLK, D), lambda i: (0, i, 0)),
            pl.BlockSpec((1, D), lambda i: (0, 0)),
        ],
        out_specs=pl.BlockSpec((_MM_BLK, D), lambda i: (i, 0)),
        out_shape=jax.ShapeDtypeStruct((N, D), jnp.float32),
    )(partials, b2)


def kernel(x, edge_index, edge_weight, W, b):
    support = _matmul(x, W)
    partials = _spmm(support, edge_index[1], edge_index[0], edge_weight)
    return _combine(partials, jnp.reshape(b, (1, D)))


# speedup vs baseline: 2.7957x; 2.7957x over previous
import functools

import jax
import jax.numpy as jnp
from jax import lax
from jax.experimental import pallas as pl
from jax.experimental.pallas import tpu as pltpu
from jax.experimental.pallas import tpu_sc as plsc

N = 10000
E = 320000
D = 128

NC = 2
NS = 16
NW = NC * NS
EPW = E // NW
CE = 80
NCH = EPW // CE
NPAD = 10240
ZCH = 16
LANES = 16
NG = CE // LANES
NLOOP = (NCH - 1) // 4

_SPLAT_DNUMS = lax.GatherDimensionNumbers(
    offset_dims=(), collapsed_slice_dims=(0,), start_index_map=(0,))

_MM_BLK = 1000


def _mm_body(x_ref, w_ref, o_ref):
    o_ref[...] = jnp.dot(x_ref[...], w_ref[...],
                         preferred_element_type=jnp.float32)


def _matmul(x, W):
    return pl.pallas_call(
        _mm_body,
        grid=(N // _MM_BLK,),
        in_specs=[
            pl.BlockSpec((_MM_BLK, D), lambda i: (i, 0)),
            pl.BlockSpec((D, D), lambda i: (0, 0)),
        ],
        out_specs=pl.BlockSpec((_MM_BLK, D), lambda i: (i, 0)),
        out_shape=jax.ShapeDtypeStruct((N, D), jnp.float32),
    )(x, W)


def _lane_splat(vec, i):
    idx = jnp.full((LANES, 1), i, dtype=jnp.int32)
    return lax.gather(vec, idx, _SPLAT_DNUMS, (1,),
                      mode=lax.GatherScatterMode.PROMISE_IN_BOUNDS)


def _spmm_body(support_hbm, src_hbm, dst_hbm, w_hbm, out_hbm,
               acc, srcs, dsts, ws, g_rows, s_rows, zbuf,
               sem_i, sem_g, sem_s, sem_z):
    c = lax.axis_index("c")
    s = lax.axis_index("s")
    wid = s * NC + c
    ebase = wid * EPW

    zv = jnp.zeros((LANES,), jnp.float32)
    for r in range(ZCH):
        for j in range(D // LANES):
            zbuf[r, pl.ds(j * LANES, LANES)] = zv

    rows_per_tile = NPAD // NS

    def _zero_fire(i, carry):
        pltpu.async_copy(zbuf,
                         acc.at[pl.ds(s * rows_per_tile + i * ZCH, ZCH)],
                         sem_z)
        return carry

    lax.fori_loop(0, rows_per_tile // ZCH, _zero_fire, 0)

    def pre(ch, k):
        sl = pl.ds(ebase + ch * CE, CE)
        pltpu.async_copy(src_hbm.at[sl], srcs.at[k], sem_i[k])
        pltpu.async_copy(dst_hbm.at[sl], dsts.at[k], sem_i[k])
        pltpu.async_copy(w_hbm.at[sl], ws.at[k, pl.ds(0, CE)], sem_i[k])

    def drain_i(ch, k):
        sl = pl.ds(ebase + ch * CE, CE)
        pltpu.make_async_copy(src_hbm.at[sl], srcs.at[k], sem_i[k]).wait()
        pltpu.make_async_copy(dst_hbm.at[sl], dsts.at[k], sem_i[k]).wait()
        pltpu.make_async_copy(w_hbm.at[sl], ws.at[k, pl.ds(0, CE)],
                              sem_i[k]).wait()

    def fire_gather(x, k):
        pltpu.async_copy(support_hbm.at[srcs.at[k]], g_rows.at[x], sem_g[x])

    def drain_gather(x, k):
        pltpu.make_async_copy(support_hbm.at[srcs.at[k]], g_rows.at[x],
                              sem_g[x]).wait()

    def fire_scatter(x, k):
        pltpu.async_copy(s_rows.at[x], acc.at[dsts.at[k]],
                         sem_s[x], add=True)

    def drain_scatter(x, k):
        pltpu.make_async_copy(s_rows.at[x], acc.at[dsts.at[k]],
                              sem_s[x]).wait()

    def scale(x, k):
        g_x = g_rows.at[x]
        s_x = s_rows.at[x]
        w_k = ws.at[k]

        def _group(g, carry):
            wv = w_k[pl.ds(g * (LANES // 2), LANES)]
            for i in range(LANES // 2):
                splat = _lane_splat(wv, i)
                r = g * (LANES // 2) + i
                for j in range(D // LANES):
                    sl = pl.ds(j * LANES, LANES)
                    s_x[r, sl] = g_x[r, sl] * splat
            return carry
        lax.fori_loop(0, 2 * NG, _group, 0)

    pre(0, 0)
    pre(1, 1)
    pre(2, 2)
    drain_i(0, 0)
    fire_gather(0, 0)
    drain_i(1, 1)
    fire_gather(1, 1)

    def _zero_drain(i, carry):
        pltpu.make_async_copy(
            zbuf,
            acc.at[pl.ds(s * rows_per_tile + i * ZCH, ZCH)], sem_z).wait()
        return carry

    lax.fori_loop(0, rows_per_tile // ZCH, _zero_drain, 0)
    plsc.subcore_barrier()

    def _quad(p, carry):
        for o in range(4):
            q = 4 * p + o
            x = o % 2
            xp = (o + 1) % 2
            k = o % 4
            kp = (k + 3) % 4
            drain_gather(x, k)
            if o == 0:
                @pl.when(p > 0)
                def _():
                    drain_scatter(xp, kp)
            else:
                drain_scatter(xp, kp)
            scale(x, k)
            if o < 2:
                pre(q + 3, kp)
            else:
                @pl.when(p < NLOOP - 1)
                def _():
                    pre(q + 3, kp)
            if o < 3:
                drain_i(q + 2, (k + 2) % 4)
                fire_gather(x, (k + 2) % 4)
            else:
                @pl.when(p < NLOOP - 1)
                def _():
                    drain_i(q + 2, (k + 2) % 4)
                    fire_gather(x, (k + 2) % 4)
            fire_scatter(x, k)
        return carry

    lax.fori_loop(0, NLOOP, _quad, 0)

    drain_gather(0, 0)
    drain_scatter(1, 3)
    scale(0, 0)
    fire_scatter(0, 0)
    drain_scatter(0, 0)
    plsc.subcore_barrier()

    out_rows = NPAD // NS
    pltpu.sync_copy(acc.at[pl.ds(s * out_rows, out_rows)],
                    out_hbm.at[c, pl.ds(s * out_rows, out_rows)])


def _spmm(support, src, dst, w):
    mesh = plsc.VectorSubcoreMesh(core_axis_name="c", subcore_axis_name="s")
    f = pl.kernel(
        _spmm_body,
        out_type=jax.ShapeDtypeStruct((NC, NPAD, D), jnp.float32),
        mesh=mesh,
        scratch_types=[
            pltpu.VMEM_SHARED((NPAD, D), jnp.float32),
            pltpu.VMEM((4, CE), jnp.int32),
            pltpu.VMEM((4, CE), jnp.int32),
            pltpu.VMEM((4, CE + 8), jnp.float32),
            pltpu.VMEM((2, CE, D), jnp.float32),
            pltpu.VMEM((2, CE, D), jnp.float32),
            pltpu.VMEM((ZCH, D), jnp.float32),
            [pltpu.SemaphoreType.DMA] * 4,
            [pltpu.SemaphoreType.DMA] * 2,
            [pltpu.SemaphoreType.DMA] * 2,
            pltpu.SemaphoreType.DMA,
        ],
    )
    return f(support, src, dst, w)


def _comb_body(p_ref, b_ref, o_ref):
    o_ref[...] = p_ref[0] + p_ref[1] + b_ref[...]


def _combine(partials, b2):
    return pl.pallas_call(
        _comb_body,
        grid=(N // _MM_BLK,),
        in_specs=[
            pl.BlockSpec((NC, _MM_BLK, D), lambda i: (0, i, 0)),
            pl.BlockSpec((1, D), lambda i: (0, 0)),
        ],
        out_specs=pl.BlockSpec((_MM_BLK, D), lambda i: (i, 0)),
        out_shape=jax.ShapeDtypeStruct((N, D), jnp.float32),
    )(partials, b2)


def kernel(x, edge_index, edge_weight, W, b):
    support = _matmul(x, W)
    partials = _spmm(support, edge_index[1], edge_index[0], edge_weight)
    return _combine(partials, jnp.reshape(b, (1, D)))


# speedup vs baseline: 2.9324x; 1.0489x over previous
import functools

import jax
import jax.numpy as jnp
from jax import lax
from jax.experimental import pallas as pl
from jax.experimental.pallas import tpu as pltpu
from jax.experimental.pallas import tpu_sc as plsc

N = 10000
E = 320000
D = 128

NC = 2
NS = 16
NW = NC * NS
EPW = E // NW
CE = 80
NCH = EPW // CE
NPAD = 10240
ZCH = 16
LANES = 16
NG = CE // LANES
NLOOP = (NCH - 1) // 4

_SPLAT_DNUMS = lax.GatherDimensionNumbers(
    offset_dims=(), collapsed_slice_dims=(0,), start_index_map=(0,))

_MM_BLK = 1000


def _lane_splat(vec, i):
    idx = jnp.full((LANES, 1), i, dtype=jnp.int32)
    return lax.gather(vec, idx, _SPLAT_DNUMS, (1,),
                      mode=lax.GatherScatterMode.PROMISE_IN_BOUNDS)


def _spmm_body(support_hbm, src_hbm, dst_hbm, w_hbm, out_hbm,
               acc, srcs, dsts, ws, g_rows, s_rows, zbuf,
               sem_i, sem_g, sem_s, sem_z):
    c = lax.axis_index("c")
    s = lax.axis_index("s")
    wid = s * NC + c
    ebase = wid * EPW

    zv = jnp.zeros((LANES,), jnp.float32)
    for r in range(ZCH):
        for j in range(D // LANES):
            zbuf[r, pl.ds(j * LANES, LANES)] = zv

    rows_per_tile = NPAD // NS

    def _zero_fire(i, carry):
        pltpu.async_copy(zbuf,
                         acc.at[pl.ds(s * rows_per_tile + i * ZCH, ZCH)],
                         sem_z)
        return carry

    lax.fori_loop(0, rows_per_tile // ZCH, _zero_fire, 0)

    def pre(ch, k):
        sl = pl.ds(ebase + ch * CE, CE)
        pltpu.async_copy(src_hbm.at[sl], srcs.at[k], sem_i[k])
        pltpu.async_copy(dst_hbm.at[sl], dsts.at[k], sem_i[k])
        pltpu.async_copy(w_hbm.at[sl], ws.at[k, pl.ds(0, CE)], sem_i[k])

    def drain_i(ch, k):
        sl = pl.ds(ebase + ch * CE, CE)
        pltpu.make_async_copy(src_hbm.at[sl], srcs.at[k], sem_i[k]).wait()
        pltpu.make_async_copy(dst_hbm.at[sl], dsts.at[k], sem_i[k]).wait()
        pltpu.make_async_copy(w_hbm.at[sl], ws.at[k, pl.ds(0, CE)],
                              sem_i[k]).wait()

    def fire_gather(x, k):
        pltpu.async_copy(support_hbm.at[srcs.at[k]], g_rows.at[x], sem_g[x])

    def drain_gather(x, k):
        pltpu.make_async_copy(support_hbm.at[srcs.at[k]], g_rows.at[x],
                              sem_g[x]).wait()

    def fire_scatter(x, k):
        pltpu.async_copy(s_rows.at[x], acc.at[dsts.at[k]],
                         sem_s[x], add=True)

    def drain_scatter(x, k):
        pltpu.make_async_copy(s_rows.at[x], acc.at[dsts.at[k]],
                              sem_s[x]).wait()

    def scale(x, k):
        g_x = g_rows.at[x]
        s_x = s_rows.at[x]
        w_k = ws.at[k]

        def _group(g, carry):
            wv = w_k[pl.ds(g * (LANES // 2), LANES)]
            for i in range(LANES // 2):
                splat = _lane_splat(wv, i)
                r = g * (LANES // 2) + i
                for j in range(D // LANES):
                    sl = pl.ds(j * LANES, LANES)
                    s_x[r, sl] = g_x[r, sl] * splat
            return carry
        lax.fori_loop(0, 2 * NG, _group, 0)

    pre(0, 0)
    pre(1, 1)
    pre(2, 2)
    drain_i(0, 0)
    fire_gather(0, 0)
    drain_i(1, 1)
    fire_gather(1, 1)

    def _zero_drain(i, carry):
        pltpu.make_async_copy(
            zbuf,
            acc.at[pl.ds(s * rows_per_tile + i * ZCH, ZCH)], sem_z).wait()
        return carry

    lax.fori_loop(0, rows_per_tile // ZCH, _zero_drain, 0)
    plsc.subcore_barrier()

    def _quad(p, carry):
        for o in range(4):
            q = 4 * p + o
            x = o % 2
            xp = (o + 1) % 2
            k = o % 4
            kp = (k + 3) % 4
            drain_gather(x, k)
            if o == 0:
                @pl.when(p > 0)
                def _():
                    drain_scatter(xp, kp)
            else:
                drain_scatter(xp, kp)
            scale(x, k)
            if o < 2:
                pre(q + 3, kp)
            else:
                @pl.when(p < NLOOP - 1)
                def _():
                    pre(q + 3, kp)
            if o < 3:
                drain_i(q + 2, (k + 2) % 4)
                fire_gather(x, (k + 2) % 4)
            else:
                @pl.when(p < NLOOP - 1)
                def _():
                    drain_i(q + 2, (k + 2) % 4)
                    fire_gather(x, (k + 2) % 4)
            fire_scatter(x, k)
        return carry

    lax.fori_loop(0, NLOOP, _quad, 0)

    drain_gather(0, 0)
    drain_scatter(1, 3)
    scale(0, 0)
    fire_scatter(0, 0)
    drain_scatter(0, 0)
    plsc.subcore_barrier()

    out_rows = NPAD // NS
    pltpu.sync_copy(acc.at[pl.ds(s * out_rows, out_rows)],
                    out_hbm.at[c, pl.ds(s * out_rows, out_rows)])


def _spmm(support, src, dst, w):
    mesh = plsc.VectorSubcoreMesh(core_axis_name="c", subcore_axis_name="s")
    f = pl.kernel(
        _spmm_body,
        out_type=jax.ShapeDtypeStruct((NC, NPAD, D), jnp.float32),
        mesh=mesh,
        scratch_types=[
            pltpu.VMEM_SHARED((NPAD, D), jnp.float32),
            pltpu.VMEM((4, CE), jnp.int32),
            pltpu.VMEM((4, CE), jnp.int32),
            pltpu.VMEM((4, CE + 8), jnp.float32),
            pltpu.VMEM((2, CE, D), jnp.float32),
            pltpu.VMEM((2, CE, D), jnp.float32),
            pltpu.VMEM((ZCH, D), jnp.float32),
            [pltpu.SemaphoreType.DMA] * 4,
            [pltpu.SemaphoreType.DMA] * 2,
            [pltpu.SemaphoreType.DMA] * 2,
            pltpu.SemaphoreType.DMA,
        ],
    )
    return f(support, src, dst, w)


def _comb_body(p_ref, w_ref, b_ref, o_ref):
    agg = p_ref[0] + p_ref[1]
    o_ref[...] = jnp.dot(agg, w_ref[...],
                         preferred_element_type=jnp.float32) + b_ref[...]


def _combine_mm(partials, W, b2):
    return pl.pallas_call(
        _comb_body,
        grid=(N // _MM_BLK,),
        in_specs=[
            pl.BlockSpec((NC, _MM_BLK, D), lambda i: (0, i, 0)),
            pl.BlockSpec((D, D), lambda i: (0, 0)),
            pl.BlockSpec((1, D), lambda i: (0, 0)),
        ],
        out_specs=pl.BlockSpec((_MM_BLK, D), lambda i: (i, 0)),
        out_shape=jax.ShapeDtypeStruct((N, D), jnp.float32),
    )(partials, W, b2)


def kernel(x, edge_index, edge_weight, W, b):
    partials = _spmm(x, edge_index[1], edge_index[0], edge_weight)
    return _combine_mm(partials, W, jnp.reshape(b, (1, D)))


# speedup vs baseline: 3.0154x; 1.0283x over previous
import functools

import jax
import jax.numpy as jnp
from jax import lax
from jax.experimental import pallas as pl
from jax.experimental.pallas import tpu as pltpu
from jax.experimental.pallas import tpu_sc as plsc

N = 10000
E = 320000
D = 128

NC = 2
NS = 16
NW = NC * NS
EPW = E // NW
CE = 80
NCH = EPW // CE
NPAD = 10240
ZCH = 16
LANES = 16
NG = CE // LANES
NLOOP = (NCH - 1) // 4

_SPLAT_DNUMS = lax.GatherDimensionNumbers(
    offset_dims=(), collapsed_slice_dims=(0,), start_index_map=(0,))

_MM_BLK = 1000


def _lane_splat(vec, i):
    idx = jnp.full((LANES, 1), i, dtype=jnp.int32)
    return lax.gather(vec, idx, _SPLAT_DNUMS, (1,),
                      mode=lax.GatherScatterMode.PROMISE_IN_BOUNDS)


def _spmm_body(support_hbm, src_hbm, dst_hbm, w_hbm, out_hbm,
               acc, srcs, dsts, ws, g_rows, s_rows, zbuf,
               sem_i, sem_g, sem_s, sem_z):
    c = lax.axis_index("c")
    s = lax.axis_index("s")
    wid = s * NC + c
    ebase = wid * EPW

    zv = jnp.zeros((LANES,), jnp.float32)
    for r in range(ZCH):
        for j in range(D // LANES):
            zbuf[r, pl.ds(j * LANES, LANES)] = zv

    rows_per_tile = NPAD // NS

    def _zero_fire(i, carry):
        pltpu.async_copy(zbuf,
                         acc.at[pl.ds(s * rows_per_tile + i * ZCH, ZCH)],
                         sem_z)
        return carry

    lax.fori_loop(0, rows_per_tile // ZCH, _zero_fire, 0)

    def pre_srcw(ch, k):
        sl = pl.ds(ebase + ch * CE, CE)
        pltpu.async_copy(src_hbm.at[sl], srcs.at[k], sem_i[k])
        pltpu.async_copy(w_hbm.at[sl], ws.at[k, pl.ds(0, CE)], sem_i[k])

    def pre_dst(ch, k):
        sl = pl.ds(ebase + ch * CE, CE)
        pltpu.async_copy(dst_hbm.at[sl], dsts.at[k], sem_i[k])

    def pre(ch, k):
        pre_srcw(ch, k)
        pre_dst(ch, k)

    def drain_i(ch, k):
        sl = pl.ds(ebase + ch * CE, CE)
        pltpu.make_async_copy(src_hbm.at[sl], srcs.at[k], sem_i[k]).wait()
        pltpu.make_async_copy(dst_hbm.at[sl], dsts.at[k], sem_i[k]).wait()
        pltpu.make_async_copy(w_hbm.at[sl], ws.at[k, pl.ds(0, CE)],
                              sem_i[k]).wait()

    def fire_gather(x, k):
        pltpu.async_copy(support_hbm.at[srcs.at[k]], g_rows.at[x], sem_g[x])

    def drain_gather(x, k):
        pltpu.make_async_copy(support_hbm.at[srcs.at[k]], g_rows.at[x],
                              sem_g[x]).wait()

    def fire_scatter(x, k):
        pltpu.async_copy(s_rows.at[x], acc.at[dsts.at[k]],
                         sem_s[x], add=True)

    def drain_scatter(x, k):
        pltpu.make_async_copy(s_rows.at[x], acc.at[dsts.at[k]],
                              sem_s[x]).wait()

    def scale(x, k):
        g_x = g_rows.at[x]
        s_x = s_rows.at[x]
        w_k = ws.at[k]

        def _group(g, carry):
            wv = w_k[pl.ds(g * (LANES // 2), LANES)]
            for i in range(LANES // 2):
                splat = _lane_splat(wv, i)
                r = g * (LANES // 2) + i
                for j in range(D // LANES):
                    sl = pl.ds(j * LANES, LANES)
                    s_x[r, sl] = g_x[r, sl] * splat
            return carry
        lax.fori_loop(0, 2 * NG, _group, 0)

    pre(0, 0)
    pre(1, 1)
    pre(2, 2)
    drain_i(0, 0)
    fire_gather(0, 0)
    drain_i(1, 1)
    fire_gather(1, 1)

    def _zero_drain(i, carry):
        pltpu.make_async_copy(
            zbuf,
            acc.at[pl.ds(s * rows_per_tile + i * ZCH, ZCH)], sem_z).wait()
        return carry

    lax.fori_loop(0, rows_per_tile // ZCH, _zero_drain, 0)
    plsc.subcore_barrier()

    def _quad(p, carry):
        for o in range(4):
            q = 4 * p + o
            x = o % 2
            xp = (o + 1) % 2
            k = o % 4
            kp = (k + 3) % 4
            drain_gather(x, k)
            if o < 2:
                pre_srcw(q + 3, kp)
            else:
                @pl.when(p < NLOOP - 1)
                def _():
                    pre_srcw(q + 3, kp)
            scale(x, k)
            if o == 0:
                @pl.when(p > 0)
                def _():
                    drain_scatter(xp, kp)
            else:
                drain_scatter(xp, kp)
            if o < 2:
                pre_dst(q + 3, kp)
            else:
                @pl.when(p < NLOOP - 1)
                def _():
                    pre_dst(q + 3, kp)
            if o < 3:
                drain_i(q + 2, (k + 2) % 4)
                fire_gather(x, (k + 2) % 4)
            else:
                @pl.when(p < NLOOP - 1)
                def _():
                    drain_i(q + 2, (k + 2) % 4)
                    fire_gather(x, (k + 2) % 4)
            fire_scatter(x, k)
        return carry

    lax.fori_loop(0, NLOOP, _quad, 0)

    drain_gather(0, 0)
    drain_scatter(1, 3)
    scale(0, 0)
    fire_scatter(0, 0)
    drain_scatter(0, 0)
    plsc.subcore_barrier()

    out_rows = NPAD // NS
    pltpu.sync_copy(acc.at[pl.ds(s * out_rows, out_rows)],
                    out_hbm.at[c, pl.ds(s * out_rows, out_rows)])


def _spmm(support, src, dst, w):
    mesh = plsc.VectorSubcoreMesh(core_axis_name="c", subcore_axis_name="s")
    f = pl.kernel(
        _spmm_body,
        out_type=jax.ShapeDtypeStruct((NC, NPAD, D), jnp.float32),
        mesh=mesh,
        scratch_types=[
            pltpu.VMEM_SHARED((NPAD, D), jnp.float32),
            pltpu.VMEM((4, CE), jnp.int32),
            pltpu.VMEM((4, CE), jnp.int32),
            pltpu.VMEM((4, CE + 8), jnp.float32),
            pltpu.VMEM((2, CE, D), jnp.float32),
            pltpu.VMEM((2, CE, D), jnp.float32),
            pltpu.VMEM((ZCH, D), jnp.float32),
            [pltpu.SemaphoreType.DMA] * 4,
            [pltpu.SemaphoreType.DMA] * 2,
            [pltpu.SemaphoreType.DMA] * 2,
            pltpu.SemaphoreType.DMA,
        ],
    )
    return f(support, src, dst, w)


def _comb_body(p_ref, w_ref, b_ref, o_ref):
    agg = p_ref[0] + p_ref[1]
    o_ref[...] = jnp.dot(agg, w_ref[...],
                         preferred_element_type=jnp.float32) + b_ref[...]


def _combine_mm(partials, W, b2):
    return pl.pallas_call(
        _comb_body,
        grid=(N // _MM_BLK,),
        in_specs=[
            pl.BlockSpec((NC, _MM_BLK, D), lambda i: (0, i, 0)),
            pl.BlockSpec((D, D), lambda i: (0, 0)),
            pl.BlockSpec((1, D), lambda i: (0, 0)),
        ],
        out_specs=pl.BlockSpec((_MM_BLK, D), lambda i: (i, 0)),
        out_shape=jax.ShapeDtypeStruct((N, D), jnp.float32),
    )(partials, W, b2)


def kernel(x, edge_index, edge_weight, W, b):
    partials = _spmm(x, edge_index[1], edge_index[0], edge_weight)
    return _combine_mm(partials, W, jnp.reshape(b, (1, D)))


# speedup vs baseline: 3.0164x; 1.0003x over previous
import jax
import jax.numpy as jnp
from jax import lax
from jax.experimental import pallas as pl
from jax.experimental.pallas import tpu as pltpu
from jax.experimental.pallas import tpu_sc as plsc

N = 10000
E = 320000
D = 128

NC = 2
NS = 16
NW = NC * NS
EPW = E // NW
CE = 80
NCH = EPW // CE
NPAD = 10240
ZCH = 16
LANES = 16
NG = CE // LANES
NLOOP = (NCH - 1) // 4

_SPLAT_DNUMS = lax.GatherDimensionNumbers(
    offset_dims=(), collapsed_slice_dims=(0,), start_index_map=(0,))

_MM_BLK = 1000


def _lane_splat(vec, i):
    idx = jnp.full((LANES, 1), i, dtype=jnp.int32)
    return lax.gather(vec, idx, _SPLAT_DNUMS, (1,),
                      mode=lax.GatherScatterMode.PROMISE_IN_BOUNDS)


def _spmm_body(support_hbm, src_hbm, dst_hbm, w_hbm, out_hbm,
               acc, srcs, dsts, ws, g_rows, s_rows, zbuf,
               sem_i, sem_g, sem_s, sem_z):
    c = lax.axis_index("c")
    s = lax.axis_index("s")
    wid = s * NC + c
    ebase = wid * EPW

    zv = jnp.zeros((LANES,), jnp.float32)
    for r in range(ZCH):
        for j in range(D // LANES):
            zbuf[r, pl.ds(j * LANES, LANES)] = zv

    rows_per_tile = NPAD // NS

    def _zero_fire(i, carry):
        pltpu.async_copy(zbuf,
                         acc.at[pl.ds(s * rows_per_tile + i * ZCH, ZCH)],
                         sem_z)
        return carry

    lax.fori_loop(0, rows_per_tile // ZCH, _zero_fire, 0)

    def pre_srcw(ch, k):
        sl = pl.ds(ebase + ch * CE, CE)
        pltpu.async_copy(src_hbm.at[sl], srcs.at[k], sem_i[k])
        pltpu.async_copy(w_hbm.at[sl], ws.at[k, pl.ds(0, CE)], sem_i[k])

    def pre_dst(ch, k):
        sl = pl.ds(ebase + ch * CE, CE)
        pltpu.async_copy(dst_hbm.at[sl], dsts.at[k], sem_i[k])

    def pre(ch, k):
        pre_srcw(ch, k)
        pre_dst(ch, k)

    def drain_i(ch, k):
        sl = pl.ds(ebase + ch * CE, CE)
        pltpu.make_async_copy(src_hbm.at[sl], srcs.at[k], sem_i[k]).wait()
        pltpu.make_async_copy(dst_hbm.at[sl], dsts.at[k], sem_i[k]).wait()
        pltpu.make_async_copy(w_hbm.at[sl], ws.at[k, pl.ds(0, CE)],
                              sem_i[k]).wait()

    def fire_gather(x, k):
        pltpu.async_copy(support_hbm.at[srcs.at[k]], g_rows.at[x], sem_g[x])

    def drain_gather(x, k):
        pltpu.make_async_copy(support_hbm.at[srcs.at[k]], g_rows.at[x],
                              sem_g[x]).wait()

    def fire_scatter(x, k):
        pltpu.async_copy(s_rows.at[x], acc.at[dsts.at[k]],
                         sem_s[x], add=True)

    def drain_scatter(x, k):
        pltpu.make_async_copy(s_rows.at[x], acc.at[dsts.at[k]],
                              sem_s[x]).wait()

    def scale(x, k):
        g_x = g_rows.at[x]
        s_x = s_rows.at[x]
        w_k = ws.at[k]

        def _group(g, carry):
            wv = w_k[pl.ds(g * (LANES // 2), LANES)]
            for i in range(LANES // 2):
                splat = _lane_splat(wv, i)
                r = g * (LANES // 2) + i
                for j in range(D // LANES):
                    sl = pl.ds(j * LANES, LANES)
                    s_x[r, sl] = g_x[r, sl] * splat
            return carry
        lax.fori_loop(0, 2 * NG, _group, 0)

    pre(0, 0)
    pre(1, 1)
    pre(2, 2)
    drain_i(0, 0)
    fire_gather(0, 0)
    drain_i(1, 1)
    fire_gather(1, 1)

    def _zero_drain(i, carry):
        pltpu.make_async_copy(
            zbuf,
            acc.at[pl.ds(s * rows_per_tile + i * ZCH, ZCH)], sem_z).wait()
        return carry

    lax.fori_loop(0, rows_per_tile // ZCH, _zero_drain, 0)
    plsc.subcore_barrier()

    def _quad(p, carry):
        for o in range(4):
            q = 4 * p + o
            x = o % 2
            xp = (o + 1) % 2
            k = o % 4
            kp = (k + 3) % 4
            drain_gather(x, k)
            if o < 2:
                pre_srcw(q + 3, kp)
            else:
                @pl.when(p < NLOOP - 1)
                def _():
                    pre_srcw(q + 3, kp)
            scale(x, k)
            if o == 0:
                @pl.when(p > 0)
                def _():
                    drain_scatter(xp, kp)
            else:
                drain_scatter(xp, kp)
            if o < 2:
                pre_dst(q + 3, kp)
            else:
                @pl.when(p < NLOOP - 1)
                def _():
                    pre_dst(q + 3, kp)
            if o < 3:
                drain_i(q + 2, (k + 2) % 4)
                fire_gather(x, (k + 2) % 4)
            else:
                @pl.when(p < NLOOP - 1)
                def _():
                    drain_i(q + 2, (k + 2) % 4)
                    fire_gather(x, (k + 2) % 4)
            fire_scatter(x, k)
        return carry

    lax.fori_loop(0, NLOOP, _quad, 0)

    drain_gather(0, 0)
    drain_scatter(1, 3)
    scale(0, 0)
    fire_scatter(0, 0)
    drain_scatter(0, 0)
    plsc.subcore_barrier()

    out_rows = NPAD // NS
    pltpu.sync_copy(acc.at[pl.ds(s * out_rows, out_rows)],
                    out_hbm.at[c, pl.ds(s * out_rows, out_rows)])


def _spmm(support, src, dst, w):
    mesh = plsc.VectorSubcoreMesh(core_axis_name="c", subcore_axis_name="s")
    f = pl.kernel(
        _spmm_body,
        out_type=jax.ShapeDtypeStruct((NC, NPAD, D), jnp.float32),
        mesh=mesh,
        scratch_types=[
            pltpu.VMEM_SHARED((NPAD, D), jnp.float32),
            pltpu.VMEM((4, CE), jnp.int32),
            pltpu.VMEM((4, CE), jnp.int32),
            pltpu.VMEM((4, CE + 8), jnp.float32),
            pltpu.VMEM((2, CE, D), jnp.float32),
            pltpu.VMEM((2, CE, D), jnp.float32),
            pltpu.VMEM((ZCH, D), jnp.float32),
            [pltpu.SemaphoreType.DMA] * 4,
            [pltpu.SemaphoreType.DMA] * 2,
            [pltpu.SemaphoreType.DMA] * 2,
            pltpu.SemaphoreType.DMA,
        ],
    )
    return f(support, src, dst, w)


def _comb_body(p_ref, w_ref, b_ref, o_ref):
    agg = p_ref[0] + p_ref[1]
    o_ref[...] = jnp.dot(agg, w_ref[...],
                         preferred_element_type=jnp.float32) + b_ref[...]


def _combine_mm(partials, W, b2):
    return pl.pallas_call(
        _comb_body,
        grid=(N // _MM_BLK,),
        in_specs=[
            pl.BlockSpec((NC, _MM_BLK, D), lambda i: (0, i, 0)),
            pl.BlockSpec((D, D), lambda i: (0, 0)),
            pl.BlockSpec((1, D), lambda i: (0, 0)),
        ],
        out_specs=pl.BlockSpec((_MM_BLK, D), lambda i: (i, 0)),
        out_shape=jax.ShapeDtypeStruct((N, D), jnp.float32),
    )(partials, W, b2)


def kernel(x, edge_index, edge_weight, W, b):
    partials = _spmm(x, edge_index[1], edge_index[0], edge_weight)
    return _combine_mm(partials, W, jnp.reshape(b, (1, D)))
